# baseline (speedup 1.0000x reference)
import jax
import jax.numpy as jnp
from jax import lax
from jax.experimental import pallas as pl
from jax.experimental.pallas import tpu as pltpu
from jax.experimental.pallas import tpu_sc as plsc

_VOCAB = 1000000
_DIM = 64
_BATCH = 16384
_HIST = 50
_NC = 2
_NS = 16
_NW = _NC * _NS
_PER_WB = _BATCH // _NW
_TB_PER_W = _PER_WB // 128
_HC = 5
_CHUNK_ROWS = _HC * 128
_HCHUNKS = _HIST // _HC
_N_CHUNKS = _TB_PER_W * _HCHUNKS
_NBUF = 2


def _gather_body(table_hbm, idx_hbm, out_hbm,
                 idx_full, idxg0, idxg1, rows0, rows1, tile0, tile1,
                 sem_g0, sem_g1, sem_t0, sem_t1):
    idx_g = (idxg0, idxg1)
    rows_v = (rows0, rows1)
    tile_v = (tile0, tile1)
    sem_g = (sem_g0, sem_g1)
    sem_t = (sem_t0, sem_t1)

    wid = lax.axis_index("s") * _NC + lax.axis_index("c")
    iota = lax.iota(jnp.int32, 16)
    vg = [iota + 16 * g for g in range(8)]

    pltpu.sync_copy(idx_hbm.at[pl.ds(pl.multiple_of(wid * _PER_WB, 8),
                                     _PER_WB)], idx_full)

    def restage(c, b):
        tb_l = c // _HCHUNKS
        h0 = (c % _HCHUNKS) * _HC
        for h_l in range(_HC):
            col = jnp.full((16,), h0 + h_l, jnp.int32)
            for g in range(8):
                vals = plsc.load_gather(idx_full, [tb_l * 128 + vg[g], col])
                idx_g[b][pl.ds(h_l * 128 + 16 * g, 16)] = vals

    def gather(c, b):
        for h_l in range(_HC):
            pltpu.async_copy(
                table_hbm.at[idx_g[b].at[pl.ds(h_l * 128, 128)]],
                rows_v[b].at[pl.ds(h_l * 128, 128)], sem_g[b])

    def wait_gather(c, b):
        for h_l in range(_HC):
            pltpu.make_async_copy(
                table_hbm.at[idx_g[b].at[pl.ds(h_l * 128, 128)]],
                rows_v[b].at[pl.ds(h_l * 128, 128)], sem_g[b]).wait()

    def tile_copy(h, td, tb, slot):
        return pltpu.make_async_copy(tile_v[slot], out_hbm.at[h, td, tb],
                                     sem_t[slot])

    def transpose_chunk(c, b):
        tb_l = c // _HCHUNKS
        h0 = (c % _HCHUNKS) * _HC
        tb = wid * _TB_PER_W + tb_l

        def h_step(h_l, carry):
            h = h0 + h_l
            row_base = h_l * 128
            rows16 = [row_base + vg[g] for g in range(8)]
            for td in range(8):
                slot = td % 2
                @pl.when((c * _HC + h_l) * 8 + td >= _NBUF)
                def _():
                    tile_copy(h, td, tb, slot).wait()
                for ds in range(8):
                    col = jnp.full((16,), td * 8 + ds, jnp.int32)
                    for g in range(8):
                        v = plsc.load_gather(rows_v[b], [rows16[g], col])
                        tile_v[slot][ds, pl.ds(16 * g, 16)] = v
                tile_copy(h, td, tb, slot).start()
            return carry

        lax.fori_loop(0, _HC, h_step, 0)

    for b in range(_NBUF):
        restage(b, b)
        gather(b, b)

    def step(g, carry):
        for b in range(_NBUF):
            c = g * _NBUF + b
            wait_gather(c, b)
            transpose_chunk(c, b)
            @pl.when(c + _NBUF < _N_CHUNKS)
            def _():
                restage(c + _NBUF, b)
                gather(c + _NBUF, b)
        return carry

    lax.fori_loop(0, _N_CHUNKS // _NBUF, step, 0)

    for slot in range(2):
        pltpu.make_async_copy(tile_v[slot], out_hbm.at[0, 0, 0],
                              sem_t[slot]).wait()


def kernel(instrument_ids, embedding_table):
    mesh = plsc.VectorSubcoreMesh(core_axis_name="c", subcore_axis_name="s")
    out = pl.kernel(
        _gather_body,
        out_type=jax.ShapeDtypeStruct((_HIST, 8, 128, 8, 128), jnp.float32),
        mesh=mesh,
        scratch_types=[
            pltpu.VMEM((_PER_WB, _HIST), jnp.int32),
            pltpu.VMEM((_CHUNK_ROWS,), jnp.int32),
            pltpu.VMEM((_CHUNK_ROWS,), jnp.int32),
            pltpu.VMEM((_CHUNK_ROWS, _DIM), jnp.float32),
            pltpu.VMEM((_CHUNK_ROWS, _DIM), jnp.float32),
            pltpu.VMEM((8, 128), jnp.float32),
            pltpu.VMEM((8, 128), jnp.float32),
            pltpu.SemaphoreType.DMA,
            pltpu.SemaphoreType.DMA,
            pltpu.SemaphoreType.DMA,
            pltpu.SemaphoreType.DMA,
        ],
        compiler_params=pltpu.CompilerParams(use_tc_tiling_on_sc=False,
                                             needs_layout_passes=False),
    )(embedding_table, instrument_ids)
    return out.transpose(2, 4, 0, 1, 3).reshape(_BATCH, _HIST, _DIM)


# speedup vs baseline: 1.1616x; 1.1616x over previous
import jax
import jax.numpy as jnp
from jax import lax
from jax.experimental import pallas as pl
from jax.experimental.pallas import tpu as pltpu
from jax.experimental.pallas import tpu_sc as plsc

_VOCAB = 1000000
_DIM = 64
_BATCH = 16384
_HIST = 50
_NC = 2
_NS = 16
_NW = _NC * _NS
_PER_WB = _BATCH // _NW
_TB_PER_W = _PER_WB // 128
_HC = 2
_CHUNK_ROWS = _HC * 128
_HCHUNKS = _HIST // _HC
_N_CHUNKS = _TB_PER_W * _HCHUNKS
_NBUF = 2


def _gather_body(table_hbm, idx_hbm, out_hbm,
                 idx_full, idxg0, idxg1, rows0, rows1, stg0, stg1,
                 sem_g0, sem_g1, sem_t0, sem_t1):
    idx_g = (idxg0, idxg1)
    rows_v = (rows0, rows1)
    stg_v = (stg0, stg1)
    sem_g = (sem_g0, sem_g1)
    sem_t = (sem_t0, sem_t1)

    wid = lax.axis_index("s") * _NC + lax.axis_index("c")
    iota = lax.iota(jnp.int32, 16)
    vg = [iota + 16 * g for g in range(8)]
    d_vec = [iota + 16 * k for k in range(4)]
    row_sc = [h_l * 8 + (d_vec[k] >> 3)
              for h_l in range(_HC) for k in range(4)]
    ds_sc = [d_vec[k] & 7 for k in range(4)]

    pltpu.sync_copy(idx_hbm.at[pl.ds(pl.multiple_of(wid * _PER_WB, 8),
                                     _PER_WB)], idx_full)

    def restage(c, b):
        tb_l = c // _HCHUNKS
        h0 = (c % _HCHUNKS) * _HC
        for h_l in range(_HC):
            col = jnp.full((16,), h0 + h_l, jnp.int32)
            for g in range(8):
                vals = plsc.load_gather(idx_full, [tb_l * 128 + vg[g], col])
                idx_g[b][pl.ds(h_l * 128 + 16 * g, 16)] = vals

    def gather(c, b):
        for h_l in range(_HC):
            pltpu.async_copy(
                table_hbm.at[idx_g[b].at[pl.ds(h_l * 128, 128)]],
                rows_v[b].at[pl.ds(h_l * 128, 128)], sem_g[b])

    def wait_gather(c, b):
        for h_l in range(_HC):
            pltpu.make_async_copy(
                table_hbm.at[idx_g[b].at[pl.ds(h_l * 128, 128)]],
                rows_v[b].at[pl.ds(h_l * 128, 128)], sem_g[b]).wait()

    def stg_copy(c, b):
        tb_l = c // _HCHUNKS
        h0 = (c % _HCHUNKS) * _HC
        tb = wid * _TB_PER_W + tb_l
        return pltpu.make_async_copy(
            stg_v[b], out_hbm.at[pl.ds(h0 * 8, _HC * 8), tb], sem_t[b])

    def transpose_chunk(b):
        for h_l in range(_HC):
            def bl_step(bl, carry):
                blv = jnp.full((16,), bl, jnp.int32)
                row = h_l * 128 + bl
                for k in range(4):
                    v = rows_v[b][row, pl.ds(16 * k, 16)]
                    plsc.store_scatter(
                        stg_v[b], [row_sc[h_l * 4 + k], ds_sc[k], blv], v)
                return carry
            lax.fori_loop(0, 128, bl_step, 0)

    for b in range(_NBUF):
        restage(b, b)
        gather(b, b)

    def step(g, carry):
        for b in range(_NBUF):
            c = g * _NBUF + b
            wait_gather(c, b)
            @pl.when(c >= _NBUF)
            def _():
                stg_copy(c - _NBUF, b).wait()
            transpose_chunk(b)
            stg_copy(c, b).start()
            @pl.when(c + _NBUF < _N_CHUNKS)
            def _():
                restage(c + _NBUF, b)
                gather(c + _NBUF, b)
        return carry

    lax.fori_loop(0, _N_CHUNKS // _NBUF, step, 0)

    for b in range(_NBUF):
        stg_copy(_N_CHUNKS - _NBUF + b, b).wait()


def kernel(instrument_ids, embedding_table):
    mesh = plsc.VectorSubcoreMesh(core_axis_name="c", subcore_axis_name="s")
    out = pl.kernel(
        _gather_body,
        out_type=jax.ShapeDtypeStruct((_HIST * 8, 128, 8, 128), jnp.float32),
        mesh=mesh,
        scratch_types=[
            pltpu.VMEM((_PER_WB, _HIST), jnp.int32),
            pltpu.VMEM((_CHUNK_ROWS,), jnp.int32),
            pltpu.VMEM((_CHUNK_ROWS,), jnp.int32),
            pltpu.VMEM((_CHUNK_ROWS, _DIM), jnp.float32),
            pltpu.VMEM((_CHUNK_ROWS, _DIM), jnp.float32),
            pltpu.VMEM((_HC * 8, 8, 128), jnp.float32),
            pltpu.VMEM((_HC * 8, 8, 128), jnp.float32),
            pltpu.SemaphoreType.DMA,
            pltpu.SemaphoreType.DMA,
            pltpu.SemaphoreType.DMA,
            pltpu.SemaphoreType.DMA,
        ],
        compiler_params=pltpu.CompilerParams(use_tc_tiling_on_sc=False,
                                             needs_layout_passes=False,
                                             disable_bounds_checks=True),
    )(embedding_table, instrument_ids)
    out = out.reshape(_HIST, 8, 128, 8, 128)
    return out.transpose(2, 4, 0, 1, 3).reshape(_BATCH, _HIST, _DIM)


# speedup vs baseline: 1.4873x; 1.2804x over previous
import jax
import jax.numpy as jnp
from jax import lax
from jax.experimental import pallas as pl
from jax.experimental.pallas import tpu as pltpu
from jax.experimental.pallas import tpu_sc as plsc

_VOCAB = 1000000
_DIM = 64
_BATCH = 16384
_HIST = 50
_TOTAL = _BATCH * _HIST
_NC = 2
_NS = 16
_NW = _NC * _NS
_PER_W = _TOTAL // _NW
_CHUNK = 800
_N_CHUNKS = _PER_W // _CHUNK
_NBUF = 2


def _gather_body(table_hbm, idx_hbm, out_hbm,
                 idx0, idx1, rows0, rows1,
                 sem_i0, sem_i1, sem_g0, sem_g1, sem_s0, sem_s1):
    idx_v = (idx0, idx1)
    rows_v = (rows0, rows1)
    sem_i = (sem_i0, sem_i1)
    sem_g = (sem_g0, sem_g1)
    sem_s = (sem_s0, sem_s1)

    wid = lax.axis_index("s") * _NC + lax.axis_index("c")
    base = wid * _PER_W

    def chunk_off(c):
        return pl.multiple_of(base + c * _CHUNK, 8)

    for b in range(_NBUF):
        pltpu.async_copy(idx_hbm.at[pl.ds(chunk_off(b), _CHUNK)],
                         idx_v[b], sem_i[b])

    def step(g, carry):
        for b in range(_NBUF):
            c = g * _NBUF + b

            @pl.when(c >= _NBUF)
            def _():
                pltpu.make_async_copy(
                    rows_v[b],
                    out_hbm.at[pl.ds(chunk_off(c - _NBUF), _CHUNK)],
                    sem_s[b]).wait()

            pltpu.make_async_copy(
                idx_hbm.at[pl.ds(chunk_off(c), _CHUNK)],
                idx_v[b], sem_i[b]).wait()

            pltpu.async_copy(table_hbm.at[idx_v[b]], rows_v[b],
                             sem_g[b]).wait()

            @pl.when(c + _NBUF < _N_CHUNKS)
            def _():
                pltpu.async_copy(
                    idx_hbm.at[pl.ds(chunk_off(c + _NBUF), _CHUNK)],
                    idx_v[b], sem_i[b])

            pltpu.async_copy(rows_v[b],
                             out_hbm.at[pl.ds(chunk_off(c), _CHUNK)],
                             sem_s[b])
        return carry

    lax.fori_loop(0, _N_CHUNKS // _NBUF, step, 0)

    for b in range(_NBUF):
        c_last = _N_CHUNKS - _NBUF + b
        pltpu.make_async_copy(rows_v[b],
                              out_hbm.at[pl.ds(chunk_off(c_last), _CHUNK)],
                              sem_s[b]).wait()


def kernel(instrument_ids, embedding_table):
    idx_flat = instrument_ids.reshape(_TOTAL)
    mesh = plsc.VectorSubcoreMesh(core_axis_name="c", subcore_axis_name="s")
    out = pl.kernel(
        _gather_body,
        out_type=jax.ShapeDtypeStruct((_TOTAL, _DIM), jnp.float32),
        mesh=mesh,
        scratch_types=[
            pltpu.VMEM((_CHUNK,), jnp.int32),
            pltpu.VMEM((_CHUNK,), jnp.int32),
            pltpu.VMEM((_CHUNK, _DIM), jnp.float32),
            pltpu.VMEM((_CHUNK, _DIM), jnp.float32),
            pltpu.SemaphoreType.DMA,
            pltpu.SemaphoreType.DMA,
            pltpu.SemaphoreType.DMA,
            pltpu.SemaphoreType.DMA,
            pltpu.SemaphoreType.DMA,
            pltpu.SemaphoreType.DMA,
        ],
        compiler_params=pltpu.CompilerParams(use_tc_tiling_on_sc=False),
    )(embedding_table, idx_flat)
    return out.reshape(_BATCH, _HIST, _DIM)
